# baseline (speedup 1.0000x reference)
import functools

import jax
import jax.numpy as jnp
from jax import lax
from jax.experimental import pallas as pl
from jax.experimental.pallas import tpu as pltpu
from jax.experimental.pallas import tpu_sc as plsc

NQ = 16
ND = 32
NCAND = 1000000
K = 100

BLK = 32768
NBLK = 31
NPAD = NBLK * BLK
CHUNK = 128
NCHUNK = NPAD // CHUNK
L2N = NCHUNK // 16
L3N = 32
P2_L2N = (K * CHUNK) // 16
P2_L2PAD = 1024
P2_L3N = 64

NEG = float("-inf")


def _score_body(q_ref, c_ref, scores_ref, cmax_ref):
    i = pl.program_id(0)
    q = q_ref[...]
    c = c_ref[...]
    s = lax.dot_general(q, c, (((1,), (1,)), ((), ())),
                        preferred_element_type=jnp.float32)
    gidx = i * BLK + lax.broadcasted_iota(jnp.int32, (NQ, BLK), 1)
    s = jnp.where(gidx < NCAND, s, NEG)
    s3 = s.reshape(NQ, BLK // CHUNK, CHUNK)
    scores_ref[...] = s3
    cmax_ref[...] = jnp.max(s3, axis=2)


def _stage_a(queries, candidates):
    return pl.pallas_call(
        _score_body,
        grid=(NBLK,),
        in_specs=[
            pl.BlockSpec((NQ, ND), lambda i: (0, 0)),
            pl.BlockSpec((BLK, ND), lambda i: (i, 0)),
        ],
        out_specs=[
            pl.BlockSpec((NQ, BLK // CHUNK, CHUNK), lambda i: (0, i, 0)),
            pl.BlockSpec((NQ, BLK // CHUNK), lambda i: (0, i)),
        ],
        out_shape=[
            jax.ShapeDtypeStruct((NQ, NCHUNK, CHUNK), jnp.float32),
            jax.ShapeDtypeStruct((NQ, NCHUNK), jnp.float32),
        ],
        compiler_params=pltpu.CompilerParams(
            dimension_semantics=("arbitrary",)),
    )(queries, candidates)


def _ffs(mask):
    return jnp.max(plsc.all_reduce_ffs(mask))


def _hmax(v):
    return lax.reduce_max(v, (0,))


def _load1(ref, i):
    return jnp.max(plsc.load_gather(ref, [jnp.full((16,), i, jnp.int32)]))


def _store1(ref, i, val, lane):
    idx = jnp.full((16,), i, jnp.int32)
    v = jnp.full((16,), val)
    plsc.store_scatter(ref, [idx], v, mask=lane == 0)


def _sel_body(scores_hbm, cmax_hbm, ids_hbm, vals_out, idx_out,
              cm_buf, l2_buf, l3_buf, chunk_ids, score_buf,
              p2l2, p2l3, val_buf, idx_buf, gath_ids, sem):
    cid = lax.axis_index("c")
    sid = lax.axis_index("s")
    q = sid
    lane = lax.iota(jnp.int32, 16)
    neg16 = jnp.full((16,), NEG, jnp.float32)

    @pl.when(cid == 0)
    def _():
        pltpu.sync_copy(cmax_hbm.at[q], cm_buf)
        l2_buf[pl.ds(L2N, 16)] = neg16
        for g in range(8):
            chunk_ids[pl.ds(g * 16, 16)] = jnp.zeros((16,), jnp.int32)
            idx_buf[pl.ds(g * 16, 16)] = jnp.zeros((16,), jnp.int32)
            val_buf[pl.ds(g * 16, 16)] = jnp.zeros((16,), jnp.float32)

        def build_l2(g, _):
            acc = neg16
            for j in range(16):
                v = plsc.load_gather(cm_buf, [lane * 16 + g * 256 + j])
                acc = jnp.maximum(acc, v)
            l2_buf[pl.ds(g * 16, 16)] = acc
            return 0

        lax.fori_loop(0, L2N // 16, build_l2, 0)

        def build_l3(g, _):
            acc = neg16
            for j in range(16):
                v = plsc.load_gather(l2_buf, [lane * 16 + g * 256 + j])
                acc = jnp.maximum(acc, v)
            l3_buf[pl.ds(g * 16, 16)] = acc
            return 0

        lax.fori_loop(0, 2, build_l3, 0)

        def extract_chunk(t, _):
            v0 = l3_buf[pl.ds(0, 16)]
            v1 = l3_buf[pl.ds(16, 16)]
            m0 = _hmax(v0)
            m1 = _hmax(v1)
            use_hi = m1 > m0
            m = jnp.maximum(m0, m1)
            grp = jnp.where(use_hi, v1, v0)
            j = jnp.where(use_hi, 16, 0) + _ffs(grp == m)
            u = l2_buf[pl.ds(j * 16, 16)]
            i_off = _ffs(u == m)
            i = j * 16 + i_off
            w = cm_buf[pl.ds(i * 16, 16)]
            c_off = _ffs(w == m)
            _store1(chunk_ids, t, i * 16 + c_off, lane)
            w2 = jnp.where(lane == c_off, NEG, w)
            cm_buf[pl.ds(i * 16, 16)] = w2
            nv = jnp.full((16,), _hmax(w2), jnp.float32)
            u2 = jnp.where(lane == i_off, nv, u)
            l2_buf[pl.ds(j * 16, 16)] = u2
            _store1(l3_buf, j, _hmax(u2), lane)
            return 0

        lax.fori_loop(0, K, extract_chunk, 0)

        pltpu.async_copy(scores_hbm.at[q].at[chunk_ids], score_buf, sem).wait()

        for g in range(P2_L2N, P2_L2PAD, 16):
            p2l2[pl.ds(g, 16)] = neg16

        def build_p2l2(g, _):
            acc = neg16
            for j in range(16):
                f = lane * 16 + g * 256 + j
                v = plsc.load_gather(score_buf, [f >> 7, f & 127])
                acc = jnp.maximum(acc, v)
            p2l2[pl.ds(g * 16, 16)] = acc
            return 0

        lax.fori_loop(0, P2_L2N // 16, build_p2l2, 0)

        def build_p2l3(g, _):
            acc = neg16
            for j in range(16):
                v = plsc.load_gather(p2l2, [lane * 16 + g * 256 + j])
                acc = jnp.maximum(acc, v)
            p2l3[pl.ds(g * 16, 16)] = acc
            return 0

        lax.fori_loop(0, 4, build_p2l3, 0)

        def extract_elem(t, _):
            v0 = p2l3[pl.ds(0, 16)]
            v1 = p2l3[pl.ds(16, 16)]
            v2 = p2l3[pl.ds(32, 16)]
            v3 = p2l3[pl.ds(48, 16)]
            m0, m1, m2, m3 = _hmax(v0), _hmax(v1), _hmax(v2), _hmax(v3)
            m = jnp.maximum(jnp.maximum(m0, m1), jnp.maximum(m2, m3))
            g = jnp.where(m0 == m, 0,
                          jnp.where(m1 == m, 1, jnp.where(m2 == m, 2, 3)))
            grp = p2l3[pl.ds(g * 16, 16)]
            j = g * 16 + _ffs(grp == m)
            u = p2l2[pl.ds(j * 16, 16)]
            i_off = _ffs(u == m)
            e = j * 16 + i_off
            row = e >> 3
            col = (e & 7) * 16
            w = score_buf[row, pl.ds(col, 16)]
            c_off = _ffs(w == m)
            f = e * 16 + c_off
            _store1(val_buf, t, m, lane)
            _store1(idx_buf, t, _load1(chunk_ids, f >> 7) * CHUNK + (f & 127), lane)
            w2 = jnp.where(lane == c_off, NEG, w)
            score_buf[row, pl.ds(col, 16)] = w2
            nv = jnp.full((16,), _hmax(w2), jnp.float32)
            u2 = jnp.where(lane == i_off, nv, u)
            p2l2[pl.ds(j * 16, 16)] = u2
            _store1(p2l3, j, _hmax(u2), lane)
            return 0

        lax.fori_loop(0, K, extract_elem, 0)

        pltpu.async_copy(ids_hbm.at[idx_buf], gath_ids, sem).wait()
        pltpu.sync_copy(val_buf, vals_out.at[q])
        pltpu.sync_copy(gath_ids, idx_out.at[q])


def _stage_b(scores3, cmax, identifiers):
    mesh = plsc.VectorSubcoreMesh(core_axis_name="c", subcore_axis_name="s")
    kfn = pl.kernel(
        _sel_body,
        out_type=[
            jax.ShapeDtypeStruct((NQ, 128), jnp.float32),
            jax.ShapeDtypeStruct((NQ, 128), jnp.int32),
        ],
        mesh=mesh,
        scratch_types=[
            pltpu.VMEM((NCHUNK,), jnp.float32),
            pltpu.VMEM((L2N + 16,), jnp.float32),
            pltpu.VMEM((L3N,), jnp.float32),
            pltpu.VMEM((128,), jnp.int32),
            pltpu.VMEM((128, CHUNK), jnp.float32),
            pltpu.VMEM((P2_L2PAD,), jnp.float32),
            pltpu.VMEM((P2_L3N,), jnp.float32),
            pltpu.VMEM((128,), jnp.float32),
            pltpu.VMEM((128,), jnp.int32),
            pltpu.VMEM((128,), jnp.int32),
            pltpu.SemaphoreType.DMA,
        ],
        compiler_params=pltpu.CompilerParams(needs_layout_passes=False),
    )
    return kfn(scores3, cmax, identifiers)


def kernel(queries, candidates, identifiers, k):
    scores3, cmax = _stage_a(queries, candidates)
    vals, idx = _stage_b(scores3, cmax, identifiers)
    return (vals[:, :K], idx[:, :K])


# speedup vs baseline: 4.4196x; 4.4196x over previous
import functools

import jax
import jax.numpy as jnp
from jax import lax
from jax.experimental import pallas as pl
from jax.experimental.pallas import tpu as pltpu
from jax.experimental.pallas import tpu_sc as plsc

NQ = 16
ND = 32
NCAND = 1000000
K = 100

BLK = 32768
NBLK = 31
NPAD = NBLK * BLK
CHUNK = 128
NCHUNK = NPAD // CHUNK
L2N = NCHUNK // 16
L3N = 32
P2_L2N = (K * CHUNK) // 16
P2_L2PAD = 1024
P2_L3N = 64

NEG = float("-inf")


def _score_body(q_ref, c_ref, scores_ref, cmax_ref):
    i = pl.program_id(0)
    q = q_ref[...]
    c = c_ref[...]
    s = lax.dot_general(q, c, (((1,), (0,)), ((), ())),
                        preferred_element_type=jnp.float32)
    gidx = i * BLK + lax.broadcasted_iota(jnp.int32, (NQ, BLK), 1)
    s = jnp.where(gidx < NCAND, s, NEG)
    s3 = s.reshape(NQ, BLK // CHUNK, CHUNK)
    scores_ref[...] = s3
    cmax_ref[...] = jnp.max(s3, axis=2)


def _stage_a(queries, candidates):
    return pl.pallas_call(
        _score_body,
        grid=(NBLK,),
        in_specs=[
            pl.BlockSpec((NQ, ND), lambda i: (0, 0)),
            pl.BlockSpec((ND, BLK), lambda i: (0, i)),
        ],
        out_specs=[
            pl.BlockSpec((NQ, BLK // CHUNK, CHUNK), lambda i: (0, i, 0)),
            pl.BlockSpec((NQ, BLK // CHUNK), lambda i: (0, i)),
        ],
        out_shape=[
            jax.ShapeDtypeStruct((NQ, NCHUNK, CHUNK), jnp.float32),
            jax.ShapeDtypeStruct((NQ, NCHUNK), jnp.float32),
        ],
        compiler_params=pltpu.CompilerParams(
            dimension_semantics=("arbitrary",)),
    )(queries, candidates)


def _ffs(mask):
    return jnp.max(plsc.all_reduce_ffs(mask))


def _hmax(v):
    return lax.reduce_max(v, (0,))


def _load1(ref, i):
    return jnp.max(plsc.load_gather(ref, [jnp.full((16,), i, jnp.int32)]))


def _store1(ref, i, val, lane):
    idx = jnp.full((16,), i, jnp.int32)
    v = jnp.full((16,), val)
    plsc.store_scatter(ref, [idx], v, mask=lane == 0)


def _sel_body(scores_hbm, cmax_hbm, ids_hbm, vals_out, idx_out,
              cm_buf, l2_buf, l3_buf, chunk_ids, score_buf,
              p2l2, p2l3, val_buf, idx_buf, gath_ids, sem):
    cid = lax.axis_index("c")
    sid = lax.axis_index("s")
    q = sid
    lane = lax.iota(jnp.int32, 16)
    neg16 = jnp.full((16,), NEG, jnp.float32)

    @pl.when(cid == 0)
    def _():
        pltpu.sync_copy(cmax_hbm.at[q], cm_buf)
        l2_buf[pl.ds(L2N, 16)] = neg16
        for g in range(8):
            chunk_ids[pl.ds(g * 16, 16)] = jnp.zeros((16,), jnp.int32)
            idx_buf[pl.ds(g * 16, 16)] = jnp.zeros((16,), jnp.int32)
            val_buf[pl.ds(g * 16, 16)] = jnp.zeros((16,), jnp.float32)

        def build_l2(g, _):
            acc = neg16
            for j in range(16):
                v = plsc.load_gather(cm_buf, [lane * 16 + g * 256 + j])
                acc = jnp.maximum(acc, v)
            l2_buf[pl.ds(g * 16, 16)] = acc
            return 0

        lax.fori_loop(0, L2N // 16, build_l2, 0)

        def build_l3(g, _):
            acc = neg16
            for j in range(16):
                v = plsc.load_gather(l2_buf, [lane * 16 + g * 256 + j])
                acc = jnp.maximum(acc, v)
            l3_buf[pl.ds(g * 16, 16)] = acc
            return 0

        lax.fori_loop(0, 2, build_l3, 0)

        def extract_chunk(t, _):
            v0 = l3_buf[pl.ds(0, 16)]
            v1 = l3_buf[pl.ds(16, 16)]
            m0 = _hmax(v0)
            m1 = _hmax(v1)
            use_hi = m1 > m0
            m = jnp.maximum(m0, m1)
            grp = jnp.where(use_hi, v1, v0)
            j = jnp.where(use_hi, 16, 0) + _ffs(grp == m)
            u = l2_buf[pl.ds(j * 16, 16)]
            i_off = _ffs(u == m)
            i = j * 16 + i_off
            w = cm_buf[pl.ds(i * 16, 16)]
            c_off = _ffs(w == m)
            _store1(chunk_ids, t, i * 16 + c_off, lane)
            w2 = jnp.where(lane == c_off, NEG, w)
            cm_buf[pl.ds(i * 16, 16)] = w2
            nv = jnp.full((16,), _hmax(w2), jnp.float32)
            u2 = jnp.where(lane == i_off, nv, u)
            l2_buf[pl.ds(j * 16, 16)] = u2
            _store1(l3_buf, j, _hmax(u2), lane)
            return 0

        lax.fori_loop(0, K, extract_chunk, 0)

        pltpu.async_copy(scores_hbm.at[q].at[chunk_ids], score_buf, sem).wait()

        for g in range(P2_L2N, P2_L2PAD, 16):
            p2l2[pl.ds(g, 16)] = neg16

        def build_p2l2(g, _):
            acc = neg16
            for j in range(16):
                f = lane * 16 + g * 256 + j
                v = plsc.load_gather(score_buf, [f >> 7, f & 127])
                acc = jnp.maximum(acc, v)
            p2l2[pl.ds(g * 16, 16)] = acc
            return 0

        lax.fori_loop(0, P2_L2N // 16, build_p2l2, 0)

        def build_p2l3(g, _):
            acc = neg16
            for j in range(16):
                v = plsc.load_gather(p2l2, [lane * 16 + g * 256 + j])
                acc = jnp.maximum(acc, v)
            p2l3[pl.ds(g * 16, 16)] = acc
            return 0

        lax.fori_loop(0, 4, build_p2l3, 0)

        def extract_elem(t, _):
            v0 = p2l3[pl.ds(0, 16)]
            v1 = p2l3[pl.ds(16, 16)]
            v2 = p2l3[pl.ds(32, 16)]
            v3 = p2l3[pl.ds(48, 16)]
            m0, m1, m2, m3 = _hmax(v0), _hmax(v1), _hmax(v2), _hmax(v3)
            m = jnp.maximum(jnp.maximum(m0, m1), jnp.maximum(m2, m3))
            g = jnp.where(m0 == m, 0,
                          jnp.where(m1 == m, 1, jnp.where(m2 == m, 2, 3)))
            grp = p2l3[pl.ds(g * 16, 16)]
            j = g * 16 + _ffs(grp == m)
            u = p2l2[pl.ds(j * 16, 16)]
            i_off = _ffs(u == m)
            e = j * 16 + i_off
            row = e >> 3
            col = (e & 7) * 16
            w = score_buf[row, pl.ds(col, 16)]
            c_off = _ffs(w == m)
            f = e * 16 + c_off
            _store1(val_buf, t, m, lane)
            _store1(idx_buf, t, _load1(chunk_ids, f >> 7) * CHUNK + (f & 127), lane)
            w2 = jnp.where(lane == c_off, NEG, w)
            score_buf[row, pl.ds(col, 16)] = w2
            nv = jnp.full((16,), _hmax(w2), jnp.float32)
            u2 = jnp.where(lane == i_off, nv, u)
            p2l2[pl.ds(j * 16, 16)] = u2
            _store1(p2l3, j, _hmax(u2), lane)
            return 0

        lax.fori_loop(0, K, extract_elem, 0)

        pltpu.async_copy(ids_hbm.at[idx_buf], gath_ids, sem).wait()
        pltpu.sync_copy(val_buf, vals_out.at[q])
        pltpu.sync_copy(gath_ids, idx_out.at[q])


def _stage_b(scores3, cmax, identifiers):
    mesh = plsc.VectorSubcoreMesh(core_axis_name="c", subcore_axis_name="s")
    kfn = pl.kernel(
        _sel_body,
        out_type=[
            jax.ShapeDtypeStruct((NQ, 128), jnp.float32),
            jax.ShapeDtypeStruct((NQ, 128), jnp.int32),
        ],
        mesh=mesh,
        scratch_types=[
            pltpu.VMEM((NCHUNK,), jnp.float32),
            pltpu.VMEM((L2N + 16,), jnp.float32),
            pltpu.VMEM((L3N,), jnp.float32),
            pltpu.VMEM((128,), jnp.int32),
            pltpu.VMEM((128, CHUNK), jnp.float32),
            pltpu.VMEM((P2_L2PAD,), jnp.float32),
            pltpu.VMEM((P2_L3N,), jnp.float32),
            pltpu.VMEM((128,), jnp.float32),
            pltpu.VMEM((128,), jnp.int32),
            pltpu.VMEM((128,), jnp.int32),
            pltpu.SemaphoreType.DMA,
        ],
        compiler_params=pltpu.CompilerParams(needs_layout_passes=False),
    )
    return kfn(scores3, cmax, identifiers)


def kernel(queries, candidates, identifiers, k):
    scores3, cmax = _stage_a(queries, candidates.T)
    vals, idx = _stage_b(scores3, cmax, identifiers)
    return (vals[:, :K], idx[:, :K])
